# baseline (speedup 1.0000x reference)
import jax
import jax.numpy as jnp
from jax import lax
from jax.experimental import pallas as pl
from jax.experimental.pallas import tpu as pltpu
from jax.experimental.pallas import tpu_sc as plsc

N = 10000
E = 320000
D = 128
G = 64

NC = 2
NS = 16
NW = NC * NS

EP = E // NW
K = 128
GCH = 20
NG = 4
EPP = NG * GCH * K
NA = N + 8
RPS = 624
ZTAIL = NA - NS * RPS
WTAIL = N - NS * RPS

RB = 320
NPAD = RB * NW

_mesh = plsc.VectorSubcoreMesh(core_axis_name="c", subcore_axis_name="s")


def _agg_body(x_hbm, src_hbm, dst_hbm, out_hbm,
              is_a, id_a, is_b, id_b, rows0, rows1, agg_sh, gsem, isem):
    c_id = lax.axis_index("c")
    s_id = lax.axis_index("s")
    wid = s_id * NC + c_id

    @pl.loop(0, K)
    def _(r):
        for j in range(D // 16):
            rows0[r, pl.ds(j * 16, 16)] = jnp.zeros((16,), jnp.float32)

    base = s_id * RPS
    for off, nrows in ((0, 128), (128, 128), (256, 128), (384, 128),
                       (512, RPS - 512)):
        pltpu.sync_copy(rows0.at[pl.ds(0, nrows)],
                        agg_sh.at[pl.ds(base + off, nrows)])

    @pl.when(s_id == NS - 1)
    def _():
        pltpu.sync_copy(rows0.at[pl.ds(0, ZTAIL)],
                        agg_sh.at[pl.ds(NS * RPS, ZTAIL)])

    def i_load(g, i_s, i_d):
        pltpu.async_copy(src_hbm.at[c_id, wid, g], i_s, isem)
        pltpu.async_copy(dst_hbm.at[wid, g], i_d, isem)

    def i_wait2():
        pltpu.make_async_copy(src_hbm.at[0, 0, 0], is_a, isem).wait()
        pltpu.make_async_copy(src_hbm.at[0, 0, 0], id_a, isem).wait()

    i_load(0, is_a, id_a)
    i_load(1, is_b, id_b)
    i_wait2()
    i_wait2()

    plsc.subcore_barrier()

    def g_start(ibuf, r, buf):
        pltpu.async_copy(x_hbm.at[ibuf.at[r]], buf, gsem)

    def g_wait(buf):
        pltpu.make_async_copy(x_hbm.at[is_a.at[0]], buf, gsem).wait()

    def scat(buf, ibuf, r):
        pltpu.sync_copy(buf, agg_sh.at[ibuf.at[r]], add=True)

    def process_group(i_s, i_d, next_fn, reload_fn):
        @pl.loop(0, GCH // 2 - 1)
        def _(k):
            c0 = 2 * k
            g_wait(rows0)
            g_start(i_s, c0 + 1, rows1)
            scat(rows0, i_d, c0)
            g_wait(rows1)
            g_start(i_s, c0 + 2, rows0)
            scat(rows1, i_d, c0 + 1)

        g_wait(rows0)
        g_start(i_s, GCH - 1, rows1)
        scat(rows0, i_d, GCH - 2)
        g_wait(rows1)
        scat(rows1, i_d, GCH - 1)
        next_fn()
        reload_fn()

    def start_next(i_s):
        def f():
            g_start(i_s, 0, rows0)
        return f

    def wait_and_start_next(i_s):
        def f():
            i_wait2()
            g_start(i_s, 0, rows0)
        return f

    def nop():
        pass

    g_start(is_a, 0, rows0)
    process_group(is_a, id_a, start_next(is_b),
                  lambda: i_load(2, is_a, id_a))
    process_group(is_b, id_b, wait_and_start_next(is_a),
                  lambda: i_load(3, is_b, id_b))
    process_group(is_a, id_a, wait_and_start_next(is_b), nop)
    process_group(is_b, id_b, nop, nop)

    plsc.subcore_barrier()

    pltpu.sync_copy(agg_sh.at[pl.ds(s_id * RPS, RPS)],
                    out_hbm.at[c_id, pl.ds(s_id * RPS, RPS)])

    @pl.when(s_id == NS - 1)
    def _():
        pltpu.sync_copy(agg_sh.at[pl.ds(NS * RPS, WTAIL)],
                        out_hbm.at[c_id, pl.ds(NS * RPS, WTAIL)])


_agg_call = pl.kernel(
    _agg_body,
    out_type=jax.ShapeDtypeStruct((NC, N, D), jnp.float32),
    mesh=_mesh,
    scratch_types=[
        pltpu.VMEM((GCH, K), jnp.int32),
        pltpu.VMEM((GCH, K), jnp.int32),
        pltpu.VMEM((GCH, K), jnp.int32),
        pltpu.VMEM((GCH, K), jnp.int32),
        pltpu.VMEM((K, D), jnp.float32),
        pltpu.VMEM((K, D), jnp.float32),
        pltpu.VMEM_SHARED((NA, D), jnp.float32),
        pltpu.SemaphoreType.DMA,
        pltpu.SemaphoreType.DMA,
    ],
)


def _mlp_block(h_ref, a_ref, w1_ref, b1_ref, w2_ref, b2_ref, o_ref):
    z = h_ref[...] + a_ref[0] + a_ref[1]
    t = jnp.dot(z, w1_ref[...], preferred_element_type=jnp.float32) + b1_ref[...]
    t = jnp.where(t > 0, t, jnp.exp(t) - 1.0)
    o_ref[...] = jnp.dot(t, w2_ref[...], preferred_element_type=jnp.float32) + b2_ref[...]


def _mlp(h, agg, W1, b1, W2, b2):
    R = 2000
    return pl.pallas_call(
        _mlp_block,
        grid=(N // R,),
        in_specs=[
            pl.BlockSpec((R, D), lambda i: (i, 0)),
            pl.BlockSpec((NC, R, D), lambda i: (0, i, 0)),
            pl.BlockSpec((D, D), lambda i: (0, 0)),
            pl.BlockSpec((1, D), lambda i: (0, 0)),
            pl.BlockSpec((D, D), lambda i: (0, 0)),
            pl.BlockSpec((1, D), lambda i: (0, 0)),
        ],
        out_specs=pl.BlockSpec((R, D), lambda i: (i, 0)),
        out_shape=jax.ShapeDtypeStruct((N, D), jnp.float32),
    )(h, agg, W1, b1.reshape(1, D), W2, b2.reshape(1, D))


def _readout_body(x_hbm, b_hbm, osum_hbm, omax_hbm,
                  xbuf, bbuf, sumbuf, maxbuf):
    c_id = lax.axis_index("c")
    s_id = lax.axis_index("s")
    wid = s_id * NC + c_id
    lo = wid * RB

    pltpu.sync_copy(x_hbm.at[pl.ds(lo, RB)], xbuf)
    pltpu.sync_copy(b_hbm.at[pl.ds(lo, RB)], bbuf)

    @pl.loop(0, G + 1)
    def _(g):
        for j in range(D // 16):
            sl = pl.ds(j * 16, 16)
            sumbuf[g, sl] = jnp.zeros((16,), jnp.float32)
            maxbuf[g, sl] = jnp.full((16,), -jnp.inf, jnp.float32)

    @pl.loop(0, RB // 16)
    def _(gg):
        bvec = bbuf[pl.ds(gg * 16, 16)]
        for jj in range(16):
            b = bvec[jj]
            i = gg * 16 + jj
            for j in range(D // 16):
                sl = pl.ds(j * 16, 16)
                v = xbuf[i, sl]
                sumbuf[b, sl] = sumbuf[b, sl] + v
                maxbuf[b, sl] = jnp.maximum(maxbuf[b, sl], v)

    pltpu.sync_copy(sumbuf.at[pl.ds(0, G)], osum_hbm.at[wid])
    pltpu.sync_copy(maxbuf.at[pl.ds(0, G)], omax_hbm.at[wid])


_readout_call = pl.kernel(
    _readout_body,
    out_type=(jax.ShapeDtypeStruct((NW, G, D), jnp.float32),
              jax.ShapeDtypeStruct((NW, G, D), jnp.float32)),
    mesh=_mesh,
    scratch_types=[
        pltpu.VMEM((RB, D), jnp.float32),
        pltpu.VMEM((RB,), jnp.int32),
        pltpu.VMEM((G + 1, D), jnp.float32),
        pltpu.VMEM((G + 1, D), jnp.float32),
    ],
)


def _final_block(ps_ref, pm_ref, w1_ref, b1_ref, w2_ref, b2_ref, o_ref):
    s = jnp.sum(ps_ref[...], axis=0)
    m = jnp.max(pm_ref[...], axis=0)
    r = jnp.concatenate([s, m], axis=-1)
    t = jnp.dot(r, w1_ref[...], preferred_element_type=jnp.float32) + b1_ref[...]
    t = jnp.where(t > 0, t, jnp.exp(t) - 1.0)
    o_ref[...] = jnp.dot(t, w2_ref[...], preferred_element_type=jnp.float32) + b2_ref[...]


def _final(ps, pm, Wp1, bp1, Wp2, bp2):
    nout = Wp2.shape[1]
    Wp2p = jnp.zeros((D, D), jnp.float32).at[:, :nout].set(Wp2)
    bp2p = jnp.zeros((1, D), jnp.float32).at[0, :nout].set(bp2)
    out = pl.pallas_call(
        _final_block,
        out_shape=jax.ShapeDtypeStruct((G, D), jnp.float32),
    )(ps, pm, Wp1, bp1.reshape(1, D), Wp2p, bp2p)
    return out[:, :nout]


def kernel(x, edge_index, batch,
           W1_0, b1_0, W2_0, b2_0,
           W1_1, b1_1, W2_1, b2_1,
           W1_2, b1_2, W2_2, b2_2,
           Wp1, bp1, Wp2, bp2):
    src1 = jnp.pad(edge_index[0].reshape(NW, EP),
                   ((0, 0), (0, EPP - EP))).reshape(NW, NG, GCH, K)
    offs = ((jnp.arange(NC)[:, None] * 8
             + (jnp.arange(NW)[None, :] // NC) // 2) * N).astype(jnp.int32)
    src = src1[None] + offs[:, :, None, None, None]
    dst = jnp.pad(edge_index[1].reshape(NW, EP),
                  ((0, 0), (0, EPP - EP)),
                  constant_values=N).reshape(NW, NG, GCH, K)

    h = x
    for (W1, b1, W2, b2) in ((W1_0, b1_0, W2_0, b2_0),
                             (W1_1, b1_1, W2_1, b2_1),
                             (W1_2, b1_2, W2_2, b2_2)):
        agg = _agg_call(jnp.concatenate([h] * 16), src, dst)
        h = _mlp(h, agg, W1, b1, W2, b2)

    hp = jnp.pad(h, ((0, NPAD - N), (0, 0)))
    bp = jnp.pad(batch, (0, NPAD - N), constant_values=G)
    ps, pm = _readout_call(hp, bp)
    return _final(ps, pm, Wp1, bp1, Wp2, bp2)


# speedup vs baseline: 1.1561x; 1.1561x over previous
import jax
import jax.numpy as jnp
from jax import lax
from jax.experimental import pallas as pl
from jax.experimental.pallas import tpu as pltpu
from jax.experimental.pallas import tpu_sc as plsc

N = 10000
E = 320000
D = 128
G = 64

NC = 2
NS = 16
NW = NC * NS
NCPY = 8

EP = E // NW
K = 128
GCH = 20
NG = 4
EPP = NG * GCH * K
NA = N + 8
RPS = 624
ZTAIL = NA - NS * RPS
WTAIL = N - NS * RPS

RB = 320
NPAD = RB * NW

_mesh = plsc.VectorSubcoreMesh(core_axis_name="c", subcore_axis_name="s")


def _agg_body(x_hbm, src_hbm, dst_hbm, out_hbm,
              is_a, id_a, is_b, id_b, rows0, rows1, agg_sh, gsem, isem):
    c_id = lax.axis_index("c")
    s_id = lax.axis_index("s")
    wid = s_id * NC + c_id

    @pl.loop(0, K)
    def _(r):
        for j in range(D // 16):
            rows0[r, pl.ds(j * 16, 16)] = jnp.zeros((16,), jnp.float32)

    base = s_id * RPS
    for off, nrows in ((0, 128), (128, 128), (256, 128), (384, 128),
                       (512, RPS - 512)):
        pltpu.sync_copy(rows0.at[pl.ds(0, nrows)],
                        agg_sh.at[pl.ds(base + off, nrows)])

    @pl.when(s_id == NS - 1)
    def _():
        pltpu.sync_copy(rows0.at[pl.ds(0, ZTAIL)],
                        agg_sh.at[pl.ds(NS * RPS, ZTAIL)])

    def i_load(g, i_s, i_d):
        pltpu.async_copy(src_hbm.at[c_id, wid, g], i_s, isem)
        pltpu.async_copy(dst_hbm.at[wid, g], i_d, isem)

    def i_wait2():
        pltpu.make_async_copy(src_hbm.at[0, 0, 0], is_a, isem).wait()
        pltpu.make_async_copy(src_hbm.at[0, 0, 0], id_a, isem).wait()

    i_load(0, is_a, id_a)
    i_load(1, is_b, id_b)
    i_wait2()
    i_wait2()

    plsc.subcore_barrier()

    def g_start(ibuf, r, buf):
        pltpu.async_copy(x_hbm.at[ibuf.at[r]], buf, gsem)

    def g_wait(buf):
        pltpu.make_async_copy(x_hbm.at[is_a.at[0]], buf, gsem).wait()

    def scat(buf, ibuf, r):
        pltpu.sync_copy(buf, agg_sh.at[ibuf.at[r]], add=True)

    def process_group(i_s, i_d, next_fn, reload_fn):
        @pl.loop(0, GCH // 2 - 1)
        def _(k):
            c0 = 2 * k
            g_wait(rows0)
            g_start(i_s, c0 + 1, rows1)
            scat(rows0, i_d, c0)
            g_wait(rows1)
            g_start(i_s, c0 + 2, rows0)
            scat(rows1, i_d, c0 + 1)

        g_wait(rows0)
        g_start(i_s, GCH - 1, rows1)
        scat(rows0, i_d, GCH - 2)
        g_wait(rows1)
        scat(rows1, i_d, GCH - 1)
        next_fn()
        reload_fn()

    def start_next(i_s):
        def f():
            g_start(i_s, 0, rows0)
        return f

    def wait_and_start_next(i_s):
        def f():
            i_wait2()
            g_start(i_s, 0, rows0)
        return f

    def nop():
        pass

    g_start(is_a, 0, rows0)
    process_group(is_a, id_a, start_next(is_b),
                  lambda: i_load(2, is_a, id_a))
    process_group(is_b, id_b, wait_and_start_next(is_a),
                  lambda: i_load(3, is_b, id_b))
    process_group(is_a, id_a, wait_and_start_next(is_b), nop)
    process_group(is_b, id_b, nop, nop)

    plsc.subcore_barrier()

    pltpu.sync_copy(agg_sh.at[pl.ds(s_id * RPS, RPS)],
                    out_hbm.at[c_id, pl.ds(s_id * RPS, RPS)])

    @pl.when(s_id == NS - 1)
    def _():
        pltpu.sync_copy(agg_sh.at[pl.ds(NS * RPS, WTAIL)],
                        out_hbm.at[c_id, pl.ds(NS * RPS, WTAIL)])


_agg_call = pl.kernel(
    _agg_body,
    out_type=jax.ShapeDtypeStruct((NC, N, D), jnp.float32),
    mesh=_mesh,
    scratch_types=[
        pltpu.VMEM((GCH, K), jnp.int32),
        pltpu.VMEM((GCH, K), jnp.int32),
        pltpu.VMEM((GCH, K), jnp.int32),
        pltpu.VMEM((GCH, K), jnp.int32),
        pltpu.VMEM((K, D), jnp.float32),
        pltpu.VMEM((K, D), jnp.float32),
        pltpu.VMEM_SHARED((NA, D), jnp.float32),
        pltpu.SemaphoreType.DMA,
        pltpu.SemaphoreType.DMA,
    ],
)


def _mlp_block_multi(h_ref, a_ref, w1_ref, b1_ref, w2_ref, b2_ref,
                     o_ref, oc_ref):
    z = h_ref[...] + a_ref[0] + a_ref[1]
    t = jnp.dot(z, w1_ref[...], preferred_element_type=jnp.float32) + b1_ref[...]
    t = jnp.where(t > 0, t, jnp.exp(t) - 1.0)
    t = jnp.dot(t, w2_ref[...], preferred_element_type=jnp.float32) + b2_ref[...]
    o_ref[...] = t
    for c in range(NCPY):
        oc_ref[c] = t


def _mlp_block(h_ref, a_ref, w1_ref, b1_ref, w2_ref, b2_ref, o_ref):
    z = h_ref[...] + a_ref[0] + a_ref[1]
    t = jnp.dot(z, w1_ref[...], preferred_element_type=jnp.float32) + b1_ref[...]
    t = jnp.where(t > 0, t, jnp.exp(t) - 1.0)
    o_ref[...] = jnp.dot(t, w2_ref[...], preferred_element_type=jnp.float32) + b2_ref[...]


def _mlp(h, agg, W1, b1, W2, b2, copies):
    R = 1000 if copies else 2000
    args = (h, agg, W1, b1.reshape(1, D), W2, b2.reshape(1, D))
    in_specs = [
        pl.BlockSpec((R, D), lambda i: (i, 0)),
        pl.BlockSpec((NC, R, D), lambda i: (0, i, 0)),
        pl.BlockSpec((D, D), lambda i: (0, 0)),
        pl.BlockSpec((1, D), lambda i: (0, 0)),
        pl.BlockSpec((D, D), lambda i: (0, 0)),
        pl.BlockSpec((1, D), lambda i: (0, 0)),
    ]
    if not copies:
        return pl.pallas_call(
            _mlp_block,
            grid=(N // R,),
            in_specs=in_specs,
            out_specs=pl.BlockSpec((R, D), lambda i: (i, 0)),
            out_shape=jax.ShapeDtypeStruct((N, D), jnp.float32),
        )(*args)
    return pl.pallas_call(
        _mlp_block_multi,
        grid=(N // R,),
        in_specs=in_specs,
        out_specs=(pl.BlockSpec((R, D), lambda i: (i, 0)),
                   pl.BlockSpec((NCPY, R, D), lambda i: (0, i, 0))),
        out_shape=(jax.ShapeDtypeStruct((N, D), jnp.float32),
                   jax.ShapeDtypeStruct((NCPY, N, D), jnp.float32)),
    )(*args)


def _readout_body(x_hbm, b_hbm, osum_hbm, omax_hbm,
                  xbuf, bbuf, sumbuf, maxbuf):
    c_id = lax.axis_index("c")
    s_id = lax.axis_index("s")
    wid = s_id * NC + c_id
    lo = wid * RB

    pltpu.sync_copy(x_hbm.at[pl.ds(lo, RB)], xbuf)
    pltpu.sync_copy(b_hbm.at[pl.ds(lo, RB)], bbuf)

    @pl.loop(0, G + 1)
    def _(g):
        for j in range(D // 16):
            sl = pl.ds(j * 16, 16)
            sumbuf[g, sl] = jnp.zeros((16,), jnp.float32)
            maxbuf[g, sl] = jnp.full((16,), -jnp.inf, jnp.float32)

    @pl.loop(0, RB // 16)
    def _(gg):
        bvec = bbuf[pl.ds(gg * 16, 16)]
        for jj in range(16):
            b = bvec[jj]
            i = gg * 16 + jj
            for j in range(D // 16):
                sl = pl.ds(j * 16, 16)
                v = xbuf[i, sl]
                sumbuf[b, sl] = sumbuf[b, sl] + v
                maxbuf[b, sl] = jnp.maximum(maxbuf[b, sl], v)

    pltpu.sync_copy(sumbuf.at[pl.ds(0, G)], osum_hbm.at[wid])
    pltpu.sync_copy(maxbuf.at[pl.ds(0, G)], omax_hbm.at[wid])


_readout_call = pl.kernel(
    _readout_body,
    out_type=(jax.ShapeDtypeStruct((NW, G, D), jnp.float32),
              jax.ShapeDtypeStruct((NW, G, D), jnp.float32)),
    mesh=_mesh,
    scratch_types=[
        pltpu.VMEM((RB, D), jnp.float32),
        pltpu.VMEM((RB,), jnp.int32),
        pltpu.VMEM((G + 1, D), jnp.float32),
        pltpu.VMEM((G + 1, D), jnp.float32),
    ],
)


def _final_block(ps_ref, pm_ref, w1_ref, b1_ref, w2_ref, b2_ref, o_ref):
    s = jnp.sum(ps_ref[...], axis=0)
    m = jnp.max(pm_ref[...], axis=0)
    r = jnp.concatenate([s, m], axis=-1)
    t = jnp.dot(r, w1_ref[...], preferred_element_type=jnp.float32) + b1_ref[...]
    t = jnp.where(t > 0, t, jnp.exp(t) - 1.0)
    o_ref[...] = jnp.dot(t, w2_ref[...], preferred_element_type=jnp.float32) + b2_ref[...]


def _final(ps, pm, Wp1, bp1, Wp2, bp2):
    nout = Wp2.shape[1]
    Wp2p = jnp.zeros((D, D), jnp.float32).at[:, :nout].set(Wp2)
    bp2p = jnp.zeros((1, D), jnp.float32).at[0, :nout].set(bp2)
    out = pl.pallas_call(
        _final_block,
        out_shape=jax.ShapeDtypeStruct((G, D), jnp.float32),
    )(ps, pm, Wp1, bp1.reshape(1, D), Wp2p, bp2p)
    return out[:, :nout]


def kernel(x, edge_index, batch,
           W1_0, b1_0, W2_0, b2_0,
           W1_1, b1_1, W2_1, b2_1,
           W1_2, b1_2, W2_2, b2_2,
           Wp1, bp1, Wp2, bp2):
    src1 = jnp.pad(edge_index[0].reshape(NW, EP),
                   ((0, 0), (0, EPP - EP))).reshape(NW, NG, GCH, K)
    offs = ((jnp.arange(NC)[:, None] * 4
             + (jnp.arange(NW)[None, :] // NC) // 4) * N).astype(jnp.int32)
    src = src1[None] + offs[:, :, None, None, None]
    dst = jnp.pad(edge_index[1].reshape(NW, EP),
                  ((0, 0), (0, EPP - EP)),
                  constant_values=N).reshape(NW, NG, GCH, K)

    h = x
    hc = jnp.concatenate([x] * NCPY)
    layers = ((W1_0, b1_0, W2_0, b2_0),
              (W1_1, b1_1, W2_1, b2_1),
              (W1_2, b1_2, W2_2, b2_2))
    for li, (W1, b1, W2, b2) in enumerate(layers):
        agg = _agg_call(hc, src, dst)
        if li < 2:
            h, hcs = _mlp(h, agg, W1, b1, W2, b2, True)
            hc = hcs.reshape(NCPY * N, D)
        else:
            h = _mlp(h, agg, W1, b1, W2, b2, False)

    hp = jnp.pad(h, ((0, NPAD - N), (0, 0)))
    bp = jnp.pad(batch, (0, NPAD - N), constant_values=G)
    ps, pm = _readout_call(hp, bp)
    return _final(ps, pm, Wp1, bp1, Wp2, bp2)
